# baseline (speedup 1.0000x reference)
import jax
import jax.numpy as jnp
from jax import lax
from jax.experimental import pallas as pl
from jax.experimental.pallas import tpu as pltpu
from jax.experimental.pallas import tpu_sc as plsc

_B = 16384
_H = 64
_L = 16
_NW = 32
_PW = _B // _NW
_C = 4
_NCH = _PW // _C
_IROWS = _PW // 128
_NK = _H // _L
_OROWS = _PW // 8


def _rescal_body(h2d, t2d, r2d, ent, rel, out_hbm,
                 hidx_v, tidx_v, ridx_v, h_rows, t_rows, rbuf, out_v,
                 sem_ht, sem_r0, sem_r1):
    wid = lax.axis_index("s") * 2 + lax.axis_index("c")

    pltpu.sync_copy(h2d.at[pl.ds(wid * _IROWS, _IROWS)], hidx_v)
    pltpu.sync_copy(t2d.at[pl.ds(wid * _IROWS, _IROWS)], tidx_v)
    pltpu.sync_copy(r2d.at[pl.ds(wid * _NCH, _NCH)], ridx_v)

    descs = []
    for j in range(_IROWS):
        descs.append(pltpu.async_copy(
            ent.at[hidx_v.at[j]], h_rows.at[pl.ds(j * 128, 128)], sem_ht))
        descs.append(pltpu.async_copy(
            ent.at[tidx_v.at[j]], t_rows.at[pl.ds(j * 128, 128)], sem_ht))
    pltpu.async_copy(rel.at[ridx_v.at[0]], rbuf.at[0], sem_r0)
    pltpu.async_copy(rel.at[ridx_v.at[1]], rbuf.at[1], sem_r1)
    for d in descs:
        d.wait()

    sems = (sem_r0, sem_r1)
    lane_iota = lax.iota(jnp.int32, _L)

    def run_chunk(chunk, b):
        rb = rbuf.at[b]
        pltpu.make_async_copy(rel.at[pl.ds(0, _C)], rb, sems[b]).wait()
        its = [chunk * _C + ci for ci in range(_C)]
        hv = [[h_rows[its[ci], pl.ds(_L * k, _L)] for k in range(_NK)]
              for ci in range(_C)]

        def lstep(lane, accs):
            idx = jnp.full((_L,), lane, dtype=jnp.int32)
            new = list(accs)
            for ci in range(_C):
                for k in range(_NK):
                    hi = hv[ci][k].at[idx].get(mode="promise_in_bounds")
                    for j in range(_NK):
                        new[_NK * ci + j] = new[_NK * ci + j] + hi * rb[
                            ci, pl.ds((k * _L + lane) * _H + _L * j, _L)]
            return tuple(new)

        zero = jnp.zeros((_L,), jnp.float32)
        accs = lax.fori_loop(0, _L, lstep, (zero,) * (_NK * _C))
        zs = []
        for ci in range(_C):
            a = accs[_NK * ci:_NK * ci + _NK]
            s = a[0] * t_rows[its[ci], pl.ds(0, _L)]
            for k in range(1, _NK):
                s = s + a[k] * t_rows[its[ci], pl.ds(_L * k, _L)]
            s = s * (-1.0 / _H)
            for sh in (1, 2, 4, 8):
                s = s + s.at[lane_iota ^ sh].get(mode="promise_in_bounds")
            zs.append(s)
        @pl.when(chunk + 2 < _NCH)
        def _refill():
            pltpu.async_copy(rel.at[ridx_v.at[chunk + 2]], rb, sems[b])
        return zs

    def outer(g, carry):
        zs = run_chunk(2 * g, 0) + run_chunk(2 * g + 1, 1)
        merged = zs[0]
        for l in range(1, 8):
            merged = jnp.where(lane_iota == l, zs[l], merged)
        out_v[g, :] = merged
        return carry

    lax.fori_loop(0, _NCH // 2, outer, 0)
    pltpu.sync_copy(out_v, out_hbm.at[pl.ds(wid * _OROWS, _OROWS)])


def _make_sc_kernel():
    mesh = plsc.VectorSubcoreMesh(core_axis_name="c", subcore_axis_name="s")
    return pl.kernel(
        _rescal_body,
        out_type=jax.ShapeDtypeStruct((_B // 8, _L), jnp.float32),
        mesh=mesh,
        compiler_params=pltpu.CompilerParams(use_tc_tiling_on_sc=False),
        scratch_types=[
            pltpu.VMEM((_IROWS, 128), jnp.int32),
            pltpu.VMEM((_IROWS, 128), jnp.int32),
            pltpu.VMEM((_NCH, _C), jnp.int32),
            pltpu.VMEM((_PW, _H), jnp.float32),
            pltpu.VMEM((_PW, _H), jnp.float32),
            pltpu.VMEM((2, _C, _H * _H), jnp.float32),
            pltpu.VMEM((_OROWS, _L), jnp.float32),
            pltpu.SemaphoreType.DMA,
            pltpu.SemaphoreType.DMA,
            pltpu.SemaphoreType.DMA,
        ],
    )


def kernel(predict_h, predict_t, predict_r, ent_embeddings, rel_matrices):
    h2d = predict_h.reshape(_B // 128, 128)
    t2d = predict_t.reshape(_B // 128, 128)
    r2d = predict_r.reshape(_B // _C, _C)
    out = _make_sc_kernel()(h2d, t2d, r2d, ent_embeddings, rel_matrices)
    return out[:, :8].reshape(_B, 1)


# speedup vs baseline: 1.9821x; 1.9821x over previous
import jax
import jax.numpy as jnp
from jax import lax
from jax.experimental import pallas as pl
from jax.experimental.pallas import tpu as pltpu
from jax.experimental.pallas import tpu_sc as plsc

_B = 16384
_H = 64
_L = 16
_NW = 32
_PW = _B // _NW
_C = 8
_NCH = _PW // _C
_NK = _H // _L


def _splat(v):
    return jnp.full((_L,), v, dtype=jnp.int32)


def _rescal_body(ih1, sh1, it1, st1, ir1, ent3, rel, out_hbm,
                 ihx, shx, itx, stx, irx, hbuf, tbuf, rbuf, out_v,
                 sem0, sem1):
    wid = lax.axis_index("s") * 2 + lax.axis_index("c")
    base = pl.multiple_of(wid * _PW, _PW)

    pltpu.sync_copy(ih1.at[pl.ds(base, _PW)], ihx)
    pltpu.sync_copy(sh1.at[pl.ds(base, _PW)], shx)
    pltpu.sync_copy(it1.at[pl.ds(base, _PW)], itx)
    pltpu.sync_copy(st1.at[pl.ds(base, _PW)], stx)
    pltpu.sync_copy(ir1.at[pl.ds(base, _PW)], irx)

    sems = (sem0, sem1)
    lane_iota = lax.iota(jnp.int32, _L)

    def fetch(chunk, b):
        off = pl.multiple_of((chunk - b) * _C, 2 * _C)
        ivh = ihx[pl.ds(off, _L)]
        ivt = itx[pl.ds(off, _L)]
        pltpu.async_copy(
            rel.at[irx.at[pl.ds(pl.multiple_of(chunk * _C, _C), _C)]],
            rbuf.at[b], sems[b])
        for i in range(_C):
            l = b * _C + i
            pltpu.async_copy(ent3.at[pl.ds(ivh[l], 1)],
                             hbuf.at[b].at[pl.ds(i, 1)], sems[b])
            pltpu.async_copy(ent3.at[pl.ds(ivt[l], 1)],
                             tbuf.at[b].at[pl.ds(i, 1)], sems[b])

    def drain(b):
        pltpu.make_async_copy(rel.at[pl.ds(0, _C)], rbuf.at[b],
                              sems[b]).wait()
        pltpu.make_async_copy(ent3.at[pl.ds(0, _C)], hbuf.at[b],
                              sems[b]).wait()
        pltpu.make_async_copy(ent3.at[pl.ds(0, _C)], tbuf.at[b],
                              sems[b]).wait()

    fetch(0, 0)
    fetch(1, 1)

    def run_chunk(chunk, b):
        drain(b)
        off = pl.multiple_of((chunk - b) * _C, 2 * _C)
        subh = shx[pl.ds(off, _L)]
        subt = stx[pl.ds(off, _L)]
        rb, hb, tb = rbuf.at[b], hbuf.at[b], tbuf.at[b]
        zs = []
        for half in (0, 1):
            zs += sub_compute_half(rb, hb, tb, subh, subt, b * _C, half)
        @pl.when(chunk + 2 < _NCH)
        def _refill():
            fetch(chunk + 2, b)
        return zs

    def sub_compute_half(rb, hb, tb, subh, subt, lane0, half):
        hv, tv = [], []
        for ci4 in range(4):
            ci = half * 4 + ci4
            sh = subh.at[_splat(lane0 + ci)].get(mode="promise_in_bounds")
            st = subt.at[_splat(lane0 + ci)].get(mode="promise_in_bounds")
            hv.append([plsc.load_gather(
                hb, [_splat(ci), sh, lane_iota + _L * k])
                for k in range(_NK)])
            tv.append([plsc.load_gather(
                tb, [_splat(ci), st, lane_iota + _L * k])
                for k in range(_NK)])

        def lstep(lane, accs):
            idx = jnp.full((_L,), lane, dtype=jnp.int32)
            new = list(accs)
            for ci4 in range(4):
                ci = half * 4 + ci4
                for k in range(_NK):
                    hi = hv[ci4][k].at[idx].get(mode="promise_in_bounds")
                    for j in range(_NK):
                        new[_NK * ci4 + j] = new[_NK * ci4 + j] + hi * rb[
                            ci, pl.ds((k * _L + lane) * _H + _L * j, _L)]
            return tuple(new)

        zero = jnp.zeros((_L,), jnp.float32)
        accs = lax.fori_loop(0, _L, lstep, (zero,) * 16)
        zs = []
        for ci4 in range(4):
            a = accs[_NK * ci4:_NK * ci4 + _NK]
            s = a[0] * tv[ci4][0]
            for k in range(1, _NK):
                s = s + a[k] * tv[ci4][k]
            s = s * (-1.0 / _H)
            for sh_ in (1, 2, 4, 8):
                s = s + s.at[lane_iota ^ sh_].get(mode="promise_in_bounds")
            zs.append(s)
        return zs

    def outer(g, carry):
        zs = run_chunk(2 * g, 0) + run_chunk(2 * g + 1, 1)
        merged = zs[0]
        for l in range(1, _L):
            merged = jnp.where(lane_iota == l, zs[l], merged)
        out_v[pl.ds(g * _L, _L)] = merged
        return carry

    lax.fori_loop(0, _NCH // 2, outer, 0)
    pltpu.sync_copy(out_v, out_hbm.at[pl.ds(base, _PW)])


def _make_sc_kernel():
    mesh = plsc.VectorSubcoreMesh(core_axis_name="c", subcore_axis_name="s")
    return pl.kernel(
        _rescal_body,
        out_type=jax.ShapeDtypeStruct((_B,), jnp.float32),
        mesh=mesh,
        compiler_params=pltpu.CompilerParams(needs_layout_passes=False),
        scratch_types=[
            pltpu.VMEM((_PW,), jnp.int32),
            pltpu.VMEM((_PW,), jnp.int32),
            pltpu.VMEM((_PW,), jnp.int32),
            pltpu.VMEM((_PW,), jnp.int32),
            pltpu.VMEM((_PW,), jnp.int32),
            pltpu.VMEM((2, _C, 8, _H), jnp.float32),
            pltpu.VMEM((2, _C, 8, _H), jnp.float32),
            pltpu.VMEM((2, _C, _H * _H), jnp.float32),
            pltpu.VMEM((_PW,), jnp.float32),
            pltpu.SemaphoreType.DMA,
            pltpu.SemaphoreType.DMA,
        ],
    )


def kernel(predict_h, predict_t, predict_r, ent_embeddings, rel_matrices):
    ih1 = predict_h // 8
    sh1 = predict_h % 8
    it1 = predict_t // 8
    st1 = predict_t % 8
    ent3 = ent_embeddings.reshape(ent_embeddings.shape[0] // 8, 8, _H)
    out = _make_sc_kernel()(ih1, sh1, it1, st1, predict_r, ent3,
                            rel_matrices)
    return out.reshape(_B, 1)
